# Initial kernel scaffold; baseline (speedup 1.0000x reference)
import jax
import jax.numpy as jnp
from jax.experimental import pallas as pl


def kernel(l_embedding, c_embedding, pos_edge_index, neg_edge_index, lW0, lb0, lW1, lb1, lW2, lb2, cW0, cb0, cW1, cb1, cW2, cb2, lu_Wih, lu_Whh, lu_bih, lu_bhh, cu_Wih, cu_Whh, cu_bih, cu_bhh):
    raise NotImplementedError("write your pallas kernel here")


# speedup vs baseline: 1.1388x; 1.1388x over previous
import functools

import jax
import jax.numpy as jnp
from jax import lax
from jax.experimental import pallas as pl
from jax.experimental.pallas import tpu as pltpu
from jax.experimental.pallas import tpu_sc as plsc

H = 128
NL = 50000
NH = 25000
NC = 20000
EP = 160000
EN = 160000
ROUNDS = 3

NTILES = 16
NSUB = 1
CHUNK = NSUB * 128

ETOT = EP + EN
CPT = 160
GCHUNKS = CPT * NTILES
GCAP = GCHUNKS * CHUNK

NC_PAD = 20224
L2C_RANGE = NC_PAD // 2
L2C_SLAB = L2C_RANGE // NTILES
C2L_OUT_PAD = 50176
C2L_RANGE = C2L_OUT_PAD // 4
C2L_SLAB = C2L_RANGE // NTILES
ACC_ROWS = C2L_RANGE + 8

BROW = 1000


def _sc_segsum_build():
    mesh = plsc.VectorSubcoreMesh(core_axis_name="c", subcore_axis_name="s")
    out_type = [
        jax.ShapeDtypeStruct((NC_PAD, H), jnp.float32),
        jax.ShapeDtypeStruct((C2L_OUT_PAD, H), jnp.float32),
    ]
    scratch = [
        pltpu.VMEM_SHARED((ACC_ROWS, H), jnp.float32),
        pltpu.VMEM((NSUB, 128), jnp.int32),
        pltpu.VMEM((NSUB, 128), jnp.int32),
        pltpu.VMEM((NSUB, 128, H), jnp.float32),
        pltpu.SemaphoreType.DMA,
    ]

    @functools.partial(pl.kernel, mesh=mesh, out_type=out_type,
                       scratch_types=scratch)
    def sc_segsum(lm, cm,
                  l2c_src0, l2c_dst0, l2c_src1, l2c_dst1,
                  c2l_src0, c2l_dst0, c2l_src1, c2l_dst1,
                  c2l_src2, c2l_dst2, c2l_src3, c2l_dst3,
                  zrows,
                  out_l2c, out_c2l,
                  acc, srcb, dstb, rowb, sem):
        cid = lax.axis_index("c")
        sid = lax.axis_index("s")

        def phase(msg, src, dst, slab, sent, out, out_base):
            pltpu.sync_copy(zrows.at[pl.ds(0, slab)],
                            acc.at[pl.ds(sid * slab, slab)])
            plsc.subcore_barrier()

            def chunk_body(k, carry):
                ch = k * NTILES + sid
                pltpu.sync_copy(src.at[ch], srcb)
                pltpu.sync_copy(dst.at[ch], dstb)
                active = dstb[0, 0:16][0] < sent

                @pl.when(active)
                def _():
                    cps = [pltpu.async_copy(msg.at[srcb.at[j]], rowb.at[j],
                                            sem)
                           for j in range(NSUB)]
                    for cp in cps:
                        cp.wait()
                    for j in range(NSUB):
                        pltpu.sync_copy(rowb.at[j], acc.at[dstb.at[j]],
                                        add=True)

                return carry

            lax.fori_loop(0, CPT, chunk_body, 0)
            plsc.subcore_barrier()
            pltpu.sync_copy(acc.at[pl.ds(sid * slab, slab)],
                            out.at[pl.ds(out_base + sid * slab, slab)])
            plsc.subcore_barrier()

        def corejob(l2c_src, l2c_dst, l2c_base, c2l_a, c2l_b):
            phase(lm, l2c_src, l2c_dst, L2C_SLAB, L2C_RANGE, out_l2c,
                  l2c_base)
            (sa, da, ba), (sb, db, bb) = c2l_a, c2l_b
            phase(cm, sa, da, C2L_SLAB, C2L_RANGE, out_c2l, ba)
            phase(cm, sb, db, C2L_SLAB, C2L_RANGE, out_c2l, bb)

        @pl.when(cid == 0)
        def _():
            corejob(l2c_src0, l2c_dst0, 0,
                    (c2l_src0, c2l_dst0, 0),
                    (c2l_src2, c2l_dst2, 2 * C2L_RANGE))

        @pl.when(cid == 1)
        def _():
            corejob(l2c_src1, l2c_dst1, L2C_RANGE,
                    (c2l_src1, c2l_dst1, C2L_RANGE),
                    (c2l_src3, c2l_dst3, 3 * C2L_RANGE))

    return sc_segsum


_sc_segsum = _sc_segsum_build()


def _dot(a, b):
    return jnp.dot(a, b, preferred_element_type=jnp.float32,
                   precision=lax.Precision.HIGHEST)


def _mlp_body(x_ref, w0, b0, w1, b1, w2, b2, out):
    x = x_ref[...]
    y = jnp.maximum(_dot(x, w0[...]) + b0[...], 0.0)
    y = jnp.maximum(_dot(y, w1[...]) + b1[...], 0.0)
    out[...] = _dot(y, w2[...]) + b2[...]


def _mlp_call(x, w0, b0, w1, b1, w2, b2):
    n = x.shape[0]
    full = lambda i: (0, 0)
    return pl.pallas_call(
        _mlp_body,
        grid=(n // BROW,),
        in_specs=[
            pl.BlockSpec((BROW, H), lambda i: (i, 0)),
            pl.BlockSpec((H, H), full), pl.BlockSpec((1, H), full),
            pl.BlockSpec((H, H), full), pl.BlockSpec((1, H), full),
            pl.BlockSpec((H, H), full), pl.BlockSpec((1, H), full),
        ],
        out_specs=pl.BlockSpec((BROW, H), lambda i: (i, 0)),
        out_shape=jax.ShapeDtypeStruct((n, H), jnp.float32),
    )(x, w0, b0, w1, b1, w2, b2)


def _gates(g, c):
    i = jax.nn.sigmoid(g[:, :H])
    f = jax.nn.sigmoid(g[:, H:2 * H])
    gg = jnp.tanh(g[:, 2 * H:3 * H])
    o = jax.nn.sigmoid(g[:, 3 * H:])
    c2 = f * c + i * gg
    return o * jnp.tanh(c2), c2


def _clstm_body(x_ref, h_ref, c_ref, wx, wh, b, h2o, c2o):
    g = _dot(x_ref[...], wx[...]) + _dot(h_ref[...], wh[...]) + b[...]
    h2, c2 = _gates(g, c_ref[...])
    h2o[...] = h2
    c2o[...] = c2


def _clstm_call(x, h, c, wx, wh, b):
    full = lambda i: (0, 0)
    row = lambda i: (i, 0)
    return pl.pallas_call(
        _clstm_body,
        grid=(NC // BROW,),
        in_specs=[
            pl.BlockSpec((BROW, H), row),
            pl.BlockSpec((BROW, H), row), pl.BlockSpec((BROW, H), row),
            pl.BlockSpec((H, 4 * H), full), pl.BlockSpec((H, 4 * H), full),
            pl.BlockSpec((1, 4 * H), full),
        ],
        out_specs=[pl.BlockSpec((BROW, H), row)] * 2,
        out_shape=[jax.ShapeDtypeStruct((NC, H), jnp.float32)] * 2,
    )(x, h, c, wx, wh, b)


def _llstm_body(x_ref, flip, h_ref, c_ref, wxa, wxb, wh, b, h2o, c2o):
    g = (_dot(x_ref[...], wxa[...]) + _dot(flip[...], wxb[...])
         + _dot(h_ref[...], wh[...]) + b[...])
    h2, c2 = _gates(g, c_ref[...])
    h2o[...] = h2
    c2o[...] = c2


def _llstm_call(x, lh, lc, wxa, wxb, wh, b):
    nblk = NL // BROW
    full = lambda i: (0, 0)
    row = lambda i: (i, 0)
    flip = lambda i: ((i + nblk // 2) % nblk, 0)
    return pl.pallas_call(
        _llstm_body,
        grid=(nblk,),
        in_specs=[
            pl.BlockSpec((BROW, H), row),
            pl.BlockSpec((BROW, H), flip),
            pl.BlockSpec((BROW, H), row), pl.BlockSpec((BROW, H), row),
            pl.BlockSpec((H, 4 * H), full), pl.BlockSpec((H, 4 * H), full),
            pl.BlockSpec((H, 4 * H), full), pl.BlockSpec((1, 4 * H), full),
        ],
        out_specs=[pl.BlockSpec((BROW, H), row)] * 2,
        out_shape=[jax.ShapeDtypeStruct((NL, H), jnp.float32)] * 2,
    )(x, lh, lh, lc, wxa, wxb, wh, b)


def _group_edges(src, dst, nranges, range_rows):
    e = src.shape[0]
    gid = dst // range_rows
    order = jnp.argsort(gid, stable=True)
    ssrc = jnp.concatenate([src[order], jnp.zeros((GCAP - e,), jnp.int32)])
    sdst = jnp.concatenate([dst[order], jnp.zeros((GCAP - e,), jnp.int32)])
    pos = jnp.arange(GCAP, dtype=jnp.int32)
    groups = []
    start = jnp.int32(0)
    for g in range(nranges):
        cnt = jnp.sum((gid == g).astype(jnp.int32))
        valid = pos < cnt
        gsrc = jnp.where(valid, jnp.roll(ssrc, -start)[:GCAP], 0)
        gdst = jnp.where(valid, jnp.roll(sdst, -start)[:GCAP] - g * range_rows,
                         range_rows)
        groups.append((gsrc.reshape(GCHUNKS, NSUB, 128),
                       gdst.reshape(GCHUNKS, NSUB, 128)))
        start = start + cnt
    return groups


def kernel(l_embedding, c_embedding, pos_edge_index, neg_edge_index,
           lW0, lb0, lW1, lb1, lW2, lb2,
           cW0, cb0, cW1, cb1, cW2, cb2,
           lu_Wih, lu_Whh, lu_bih, lu_bhh,
           cu_Wih, cu_Whh, cu_bih, cu_bhh):
    pos_src = pos_edge_index[0].astype(jnp.int32)
    pos_dst = pos_edge_index[1].astype(jnp.int32)
    neg_src = neg_edge_index[0].astype(jnp.int32)
    neg_dst = neg_edge_index[1].astype(jnp.int32)

    l2c = _group_edges(jnp.concatenate([pos_src, neg_src + NH]),
                       jnp.concatenate([pos_dst, neg_dst]), 2, L2C_RANGE)
    c2l = _group_edges(jnp.concatenate([pos_dst, neg_dst]),
                       jnp.concatenate([pos_src, neg_src + NH]), 4, C2L_RANGE)
    zrows = jnp.zeros((C2L_SLAB, H), jnp.float32)

    lw = [lW0.T, lb0.reshape(1, H), lW1.T, lb1.reshape(1, H),
          lW2.T, lb2.reshape(1, H)]
    cw = [cW0.T, cb0.reshape(1, H), cW1.T, cb1.reshape(1, H),
          cW2.T, cb2.reshape(1, H)]
    cu_wx = cu_Wih.T
    cu_wh = cu_Whh.T
    cu_b = (cu_bih + cu_bhh).reshape(1, 4 * H)
    lu_wxa = lu_Wih[:, :H].T
    lu_wxb = lu_Wih[:, H:].T
    lu_wh = lu_Whh.T
    lu_b = (lu_bih + lu_bhh).reshape(1, 4 * H)

    l_h = l_embedding
    l_c = jnp.zeros_like(l_embedding)
    c_h = c_embedding
    c_c = jnp.zeros_like(c_embedding)

    for _ in range(ROUNDS):
        lm = _mlp_call(l_h, *lw)
        cm = _mlp_call(c_h, *cw)
        l2c_msg, c2l_msg = _sc_segsum(
            lm, cm,
            l2c[0][0], l2c[0][1], l2c[1][0], l2c[1][1],
            c2l[0][0], c2l[0][1], c2l[1][0], c2l[1][1],
            c2l[2][0], c2l[2][1], c2l[3][0], c2l[3][1],
            zrows)
        new_ch, new_cc = _clstm_call(l2c_msg, c_h, c_c, cu_wx, cu_wh, cu_b)
        new_lh, new_lc = _llstm_call(c2l_msg, l_h, l_c,
                                     lu_wxa, lu_wxb, lu_wh, lu_b)
        c_h, c_c = new_ch, new_cc
        l_h, l_c = new_lh, new_lc

    return (l_h, c_h)
